# baseline (speedup 1.0000x reference)
import functools

import jax
import jax.numpy as jnp
from jax.experimental import pallas as pl
from jax.experimental.pallas import tpu as pltpu

N_NODES = 10000
N_EDGES = 320000
D = 128
ED = 16

EDGE_BLK = 2560
NODE_BLK = 2000


def _silu(v):
    return v * jax.nn.sigmoid(v)


def _edge_body(d_ref, hs_ref, hd_ref, xs_ref, xd_ref,
               w1s_ref, w1d_ref, w1e_ref, b1_ref, wc2_ref,
               we1_ref, be1_ref, we2_ref, be2_ref,
               m_ref, cu_ref):
    f32 = jnp.float32
    d = d_ref[...]
    ea = _silu(d * we1_ref[0:1, :] + be1_ref[0:1, :])
    ea = jnp.dot(ea, we2_ref[...], preferred_element_type=f32) + be2_ref[0:1, :]
    act = (jnp.dot(hs_ref[...], w1s_ref[...], preferred_element_type=f32)
           + jnp.dot(hd_ref[...], w1d_ref[...], preferred_element_type=f32)
           + jnp.dot(ea, w1e_ref[...], preferred_element_type=f32)
           + b1_ref[0:1, :])
    s = _silu(act)
    m_ref[...] = s[:, :D]
    w = jnp.dot(s[:, D:], wc2_ref[...], preferred_element_type=f32)
    xdiff = xs_ref[...] - xd_ref[...]
    s2 = jnp.sum(xdiff * xdiff, axis=1, keepdims=True)
    ln = jnp.maximum(jnp.sqrt(s2), 1e-8)
    cu = (w / ln) * xdiff
    lane = jax.lax.broadcasted_iota(jnp.int32, cu.shape, 1)
    cu_ref[...] = jnp.where(lane == 3, 1.0, cu)


def _edge_compute(d2, hs, hd, xs, xd, w1s, w1d, w1e, b1, wc2, we1, be1, we2, be2):
    nb = N_EDGES // EDGE_BLK
    be = EDGE_BLK
    ew = lambda: pl.BlockSpec((be, None), lambda i: (i, 0))
    full = lambda a: pl.BlockSpec(a.shape, lambda i: (0,) * a.ndim)
    grid_spec = pl.GridSpec(
        grid=(nb,),
        in_specs=[
            pl.BlockSpec((be, 1), lambda i: (i, 0)),
            pl.BlockSpec((be, D), lambda i: (i, 0)),
            pl.BlockSpec((be, D), lambda i: (i, 0)),
            pl.BlockSpec((be, 16), lambda i: (i, 0)),
            pl.BlockSpec((be, 16), lambda i: (i, 0)),
            full(w1s), full(w1d), full(w1e), full(b1), full(wc2),
            full(we1), full(be1), full(we2), full(be2),
        ],
        out_specs=[
            pl.BlockSpec((be, D), lambda i: (i, 0)),
            pl.BlockSpec((be, 16), lambda i: (i, 0)),
        ],
    )
    return pl.pallas_call(
        _edge_body,
        grid_spec=grid_spec,
        out_shape=[
            jax.ShapeDtypeStruct((N_EDGES, D), jnp.float32),
            jax.ShapeDtypeStruct((N_EDGES, 16), jnp.float32),
        ],
    )(d2, hs, hd, xs, xd, w1s, w1d, w1e, b1, wc2, we1, be1, we2, be2)


def _finish_body(h_ref, x_ref, p0_ref, p1_ref, q0_ref, q1_ref,
                 wn2_ref, bn2_ref, ho_ref, xo_ref):
    hs = p0_ref[...] + p1_ref[...]
    xs = q0_ref[...] + q1_ref[...]
    deg = xs[:, 3:4]
    ho_ref[...] = (h_ref[...]
                   + jnp.dot(hs, wn2_ref[...], preferred_element_type=jnp.float32)
                   + deg * bn2_ref[0:1, :])
    xo_ref[...] = x_ref[...] + xs[:, :3]


def _finish(h, x, p0, p1, q0, q1, wn2, bn2):
    nb = N_NODES // NODE_BLK
    b = NODE_BLK
    full = lambda a: pl.BlockSpec(a.shape, lambda i: (0,) * a.ndim)
    grid_spec = pl.GridSpec(
        grid=(nb,),
        in_specs=[
            pl.BlockSpec((b, D), lambda i: (i, 0)),
            pl.BlockSpec((b, 3), lambda i: (i, 0)),
            pl.BlockSpec((b, D), lambda i: (i, 0)),
            pl.BlockSpec((b, D), lambda i: (i, 0)),
            pl.BlockSpec((b, 16), lambda i: (i, 0)),
            pl.BlockSpec((b, 16), lambda i: (i, 0)),
            full(wn2), full(bn2),
        ],
        out_specs=[
            pl.BlockSpec((b, D), lambda i: (i, 0)),
            pl.BlockSpec((b, 3), lambda i: (i, 0)),
        ],
    )
    return pl.pallas_call(
        _finish_body,
        grid_spec=grid_spec,
        out_shape=[
            jax.ShapeDtypeStruct((N_NODES, D), jnp.float32),
            jax.ShapeDtypeStruct((N_NODES, 3), jnp.float32),
        ],
    )(h, x, p0, p1, q0, q1, wn2, bn2)


def kernel(h, x, edge_index, edge_dist, We1, be1, We2, be2,
           Wn1, bn1, Wn2, bn2, Wc1, bc1, Wc2):
    src = edge_index[0]
    dst = edge_index[1]
    f32 = jnp.float32

    w1s = jnp.concatenate([Wn1[:D], Wc1[:D]], axis=1)
    w1d = jnp.concatenate([Wn1[D:2 * D], Wc1[D:2 * D]], axis=1)
    w1e = jnp.concatenate([Wn1[2 * D:], Wc1[2 * D:]], axis=1)
    b1 = jnp.pad(jnp.concatenate([bn1, bc1])[None, :], ((0, 7), (0, 0)))
    we1p = jnp.pad(We1, ((0, 7), (0, 0)))
    be1p = jnp.pad(be1[None, :], ((0, 7), (0, 0)))
    be2p = jnp.pad(be2[None, :], ((0, 7), (0, 0)))
    bn2p = jnp.pad(bn2[None, :], ((0, 7), (0, 0)))

    xp = jnp.pad(x, ((0, 0), (0, 13)))

    hs = jnp.take(h, src, axis=0)
    hd = jnp.take(h, dst, axis=0)
    xs = jnp.take(xp, src, axis=0)
    xd = jnp.take(xp, dst, axis=0)

    d2 = edge_dist[:, None].astype(f32)
    m, cu = _edge_compute(d2, hs, hd, xs, xd, w1s, w1d, w1e, b1, Wc2,
                          we1p, be1p, We2, be2p)

    half = N_EDGES // 2
    p0 = jax.ops.segment_sum(m[:half], dst[:half], num_segments=N_NODES)
    p1 = jax.ops.segment_sum(m[half:], dst[half:], num_segments=N_NODES)
    q0 = jax.ops.segment_sum(cu[:half], dst[:half], num_segments=N_NODES)
    q1 = jax.ops.segment_sum(cu[half:], dst[half:], num_segments=N_NODES)

    ho, xo = _finish(h, x, p0, p1, q0, q1, Wn2, bn2p)
    return (ho, xo)
